# baseline (speedup 1.0000x reference)
import functools

import jax
import jax.numpy as jnp
import numpy as np
from jax import lax
from jax.experimental import pallas as pl
from jax.experimental.pallas import tpu as pltpu
from jax.experimental.pallas import tpu_sc as plsc

N = 10000
NP = 10240
D = 128
NC = 2
NS = 16
NW = NC * NS
E = 320000
C = 80
ROWS_PT = NP // NS


NCHR = 250
NCHL = 252


def _sc_spmm_body(x_hbm, alpha_hbm, slp, tlp, vlp, shp, thp, vhp, out_hbm,
                  rows0, rows1, rows2, rows3, sbuf, tbuf, vbuf, albuf, dbuf,
                  scale_v, sem_g, sem_e, sem_a, sem_s, z_sh, alpha_sh):
    cid = lax.axis_index("c")
    sid = lax.axis_index("s")
    rows = (rows0, rows1, rows2, rows3)

    @pl.when(sid == 0)
    def _():
        pltpu.sync_copy(alpha_hbm, alpha_sh)

    plsc.subcore_barrier()

    start = sid * ROWS_PT

    def _main(src_hbm, dst_hbm, val_hbm, lp):
        def _stage(k, slot, sem):
            pltpu.async_copy(src_hbm.at[sid, k], sbuf.at[slot], sem)
            pltpu.async_copy(dst_hbm.at[sid, k], tbuf.at[slot], sem)
            pltpu.async_copy(val_hbm.at[sid, k], vbuf.at[slot], sem)

        def _stage_wait(k, slot, sem):
            pltpu.make_async_copy(
                src_hbm.at[sid, k], sbuf.at[slot], sem).wait()
            pltpu.make_async_copy(
                dst_hbm.at[sid, k], tbuf.at[slot], sem).wait()
            pltpu.make_async_copy(
                val_hbm.at[sid, k], vbuf.at[slot], sem).wait()

        for k in range(3):
            _stage(k, k, sem_e)
            _stage_wait(k, k, sem_e)
            pltpu.async_copy(x_hbm.at[sbuf.at[k]], rows[k], sem_g)
            pltpu.async_copy(alpha_sh.at[tbuf.at[k]], albuf.at[k], sem_a)
        _stage(3, 3, sem_e)

        zero = jnp.zeros((16,), jnp.float32)

        def _zrow(e, carry):
            for v in range(D // 16):
                rows3[e, pl.ds(v * 16, 16)] = zero
            return carry

        lax.fori_loop(0, C, _zrow, 0)
        for c in range(ROWS_PT // C):
            pltpu.sync_copy(rows3, z_sh.at[pl.ds(start + c * C, C)])

        plsc.subcore_barrier()

        def _step(j, u):
            nx = (u + 3) % 4

            @pl.when(j < NCHR)
            def _():
                pltpu.make_async_copy(
                    x_hbm.at[sbuf.at[u]], rows[u], sem_g).wait()
                pltpu.make_async_copy(
                    alpha_sh.at[tbuf.at[u]], albuf.at[u], sem_a).wait()

            @pl.when((j > 0) & (j < NCHR + 1))
            def _():
                pltpu.make_async_copy(
                    rows[nx], z_sh.at[dbuf.at[nx]], sem_s).wait()

            @pl.when(j + 3 < NCHR)
            def _():
                _stage_wait(j + 3, nx, sem_e)
                pltpu.async_copy(x_hbm.at[sbuf.at[nx]], rows[nx], sem_g)
                pltpu.async_copy(
                    alpha_sh.at[tbuf.at[nx]], albuf.at[nx], sem_a)

            @pl.when(j < NCHR)
            def _():
                for g in range(C // 16):
                    sl = pl.ds(g * 16, 16)
                    av = albuf[u, sl]
                    scale_v[sl] = vbuf[u, sl] * (av if lp else 1.0 - av)
                    dbuf[u, sl] = tbuf[u, sl]

            @pl.when(j + 4 < NCHR)
            def _():
                _stage(j + 4, u, sem_e)

            @pl.when(j < NCHR)
            def _():
                def _erow(e, carry):
                    s16 = plsc.load_gather(
                        scale_v, [jnp.full((16,), e, jnp.int32)])
                    for v in range(D // 16):
                        sl = pl.ds(v * 16, 16)
                        rows[u][e, sl] = rows[u][e, sl] * s16
                    return carry

                lax.fori_loop(0, C, _erow, 0, unroll=8)

                pltpu.async_copy(rows[u], z_sh.at[dbuf.at[u]], sem_s, add=True)

        def _outer(jj, carry):
            for u in range(4):
                _step(jj * 4 + u, u)
            return carry

        lax.fori_loop(0, NCHL // 4, _outer, 0)

    @pl.when(cid == 0)
    def _():
        _main(slp, tlp, vlp, True)

    @pl.when(cid == 1)
    def _():
        _main(shp, thp, vhp, False)

    plsc.subcore_barrier()

    for c in range(ROWS_PT // C):
        r0 = start + c * C
        pltpu.sync_copy(z_sh.at[pl.ds(r0, C)], out_hbm.at[cid, pl.ds(r0, C)])


_sc_spmm = functools.partial(
    pl.kernel,
    out_type=jax.ShapeDtypeStruct((NC, NP, D), jnp.float32),
    mesh=plsc.VectorSubcoreMesh(core_axis_name="c", subcore_axis_name="s",
                                num_cores=NC, num_subcores=NS),
    compiler_params=pltpu.CompilerParams(needs_layout_passes=False),
    scratch_types=[
        pltpu.VMEM((C, D), jnp.float32),
        pltpu.VMEM((C, D), jnp.float32),
        pltpu.VMEM((C, D), jnp.float32),
        pltpu.VMEM((C, D), jnp.float32),
        pltpu.VMEM((4, C), jnp.int32),
        pltpu.VMEM((4, C), jnp.int32),
        pltpu.VMEM((4, C), jnp.float32),
        pltpu.VMEM((4, C), jnp.float32),
        pltpu.VMEM((4, C), jnp.int32),
        pltpu.VMEM((C,), jnp.float32),
        pltpu.SemaphoreType.DMA,
        pltpu.SemaphoreType.DMA,
        pltpu.SemaphoreType.DMA,
        pltpu.SemaphoreType.DMA,
        pltpu.VMEM_SHARED((NP, D), jnp.float32),
        pltpu.VMEM_SHARED((N,), jnp.float32),
    ],
)(_sc_spmm_body)


def _alpha_body(x_ref, tw_ref, tb_ref, o_ref):
    t = jnp.sum(x_ref[...] * tw_ref[...], axis=1, keepdims=True) + tb_ref[0, 0]
    o_ref[...] = 1.0 / (1.0 + jnp.exp(-t))


def _alpha_tc(x, theta_w, theta_b):
    blk = 400
    return pl.pallas_call(
        _alpha_body,
        grid=(N // blk,),
        in_specs=[
            pl.BlockSpec((blk, D), lambda i: (i, 0)),
            pl.BlockSpec((1, D), lambda i: (0, 0)),
            pl.BlockSpec((1, 1), lambda i: (0, 0)),
        ],
        out_specs=pl.BlockSpec((blk, 1), lambda i: (i, 0)),
        out_shape=jax.ShapeDtypeStruct((N, 1), jnp.float32),
    )(x, theta_w.reshape(1, D), theta_b.reshape(1, 1))


def _out_body(p_ref, w_ref, b_ref, o_ref):
    z = p_ref[0] + p_ref[1]
    o_ref[...] = jnp.maximum(
        jnp.dot(z, w_ref[...], preferred_element_type=jnp.float32) + b_ref[...],
        0.0)


def _out_tc(parts, W, b):
    blk = 2000
    return pl.pallas_call(
        _out_body,
        grid=(N // blk,),
        in_specs=[
            pl.BlockSpec((NC, blk, D), lambda i: (0, i, 0)),
            pl.BlockSpec((D, D), lambda i: (0, 0)),
            pl.BlockSpec((1, D), lambda i: (0, 0)),
        ],
        out_specs=pl.BlockSpec((blk, D), lambda i: (i, 0)),
        out_shape=jax.ShapeDtypeStruct((N, D), jnp.float32),
    )(parts, W, b.reshape(1, D))


def kernel(x, theta_w, theta_b, W, b, vals_lp, src_lp, dst_lp,
           vals_hp, src_hp, dst_hp):
    alpha = _alpha_tc(x, theta_w, theta_b)
    shp3 = (NS, NCHR, C)
    parts = _sc_spmm(x, alpha.reshape(N),
                     src_lp.reshape(shp3), dst_lp.reshape(shp3),
                     vals_lp.reshape(shp3),
                     src_hp.reshape(shp3), dst_hp.reshape(shp3),
                     vals_hp.reshape(shp3))
    out = _out_tc(parts, W, b)
    return out, alpha


# speedup vs baseline: 1.0006x; 1.0006x over previous
import functools

import jax
import jax.numpy as jnp
from jax import lax
from jax.experimental import pallas as pl
from jax.experimental.pallas import tpu as pltpu
from jax.experimental.pallas import tpu_sc as plsc

N = 10000
NP = 10240
D = 128
NC = 2
NS = 16
C = 80
ROWS_PT = NP // NS


NCHR = 250
NCHL = 252


def _sc_spmm_body(x_hbm, alpha_hbm, slp, tlp, vlp, shp, thp, vhp, out_hbm,
                  rows0, rows1, rows2, rows3, sbuf, tbuf, vbuf, albuf, dbuf,
                  scale_v, sem_g, sem_e, sem_a, sem_s, z_sh, alpha_sh):
    cid = lax.axis_index("c")
    sid = lax.axis_index("s")
    rows = (rows0, rows1, rows2, rows3)

    @pl.when(sid == 0)
    def _():
        pltpu.sync_copy(alpha_hbm, alpha_sh)

    plsc.subcore_barrier()

    start = sid * ROWS_PT

    def _main(src_hbm, dst_hbm, val_hbm, lp):
        def _stage(k, slot, sem):
            pltpu.async_copy(src_hbm.at[sid, k], sbuf.at[slot], sem)
            pltpu.async_copy(dst_hbm.at[sid, k], tbuf.at[slot], sem)
            pltpu.async_copy(val_hbm.at[sid, k], vbuf.at[slot], sem)

        def _stage_wait(k, slot, sem):
            pltpu.make_async_copy(
                src_hbm.at[sid, k], sbuf.at[slot], sem).wait()
            pltpu.make_async_copy(
                dst_hbm.at[sid, k], tbuf.at[slot], sem).wait()
            pltpu.make_async_copy(
                val_hbm.at[sid, k], vbuf.at[slot], sem).wait()

        for k in range(3):
            _stage(k, k, sem_e)
            _stage_wait(k, k, sem_e)
            pltpu.async_copy(x_hbm.at[sbuf.at[k]], rows[k], sem_g)
            pltpu.async_copy(alpha_sh.at[tbuf.at[k]], albuf.at[k], sem_a)
        _stage(3, 3, sem_e)

        zero = jnp.zeros((16,), jnp.float32)

        def _zrow(e, carry):
            for v in range(D // 16):
                rows3[e, pl.ds(v * 16, 16)] = zero
            return carry

        lax.fori_loop(0, C, _zrow, 0)
        for c in range(ROWS_PT // C):
            pltpu.sync_copy(rows3, z_sh.at[pl.ds(start + c * C, C)])

        plsc.subcore_barrier()

        def _step(j, u):
            nx = (u + 3) % 4

            @pl.when(j < NCHR)
            def _():
                pltpu.make_async_copy(
                    x_hbm.at[sbuf.at[u]], rows[u], sem_g).wait()
                pltpu.make_async_copy(
                    alpha_sh.at[tbuf.at[u]], albuf.at[u], sem_a).wait()

            @pl.when((j > 0) & (j < NCHR + 1))
            def _():
                pltpu.make_async_copy(
                    rows[nx], z_sh.at[dbuf.at[nx]], sem_s).wait()

            @pl.when(j + 3 < NCHR)
            def _():
                _stage_wait(j + 3, nx, sem_e)
                pltpu.async_copy(x_hbm.at[sbuf.at[nx]], rows[nx], sem_g)
                pltpu.async_copy(
                    alpha_sh.at[tbuf.at[nx]], albuf.at[nx], sem_a)

            @pl.when(j < NCHR)
            def _():
                for g in range(C // 16):
                    sl = pl.ds(g * 16, 16)
                    av = albuf[u, sl]
                    scale_v[sl] = vbuf[u, sl] * (av if lp else 1.0 - av)
                    dbuf[u, sl] = tbuf[u, sl]

            @pl.when(j + 4 < NCHR)
            def _():
                _stage(j + 4, u, sem_e)

            @pl.when(j < NCHR)
            def _():
                def _erow(e, carry):
                    s16 = plsc.load_gather(
                        scale_v, [jnp.full((16,), e, jnp.int32)])
                    for v in range(D // 16):
                        sl = pl.ds(v * 16, 16)
                        rows[u][e, sl] = rows[u][e, sl] * s16
                    return carry

                lax.fori_loop(0, C, _erow, 0, unroll=8)

                pltpu.async_copy(rows[u], z_sh.at[dbuf.at[u]], sem_s, add=True)

        def _outer(jj, carry):
            for u in range(4):
                _step(jj * 4 + u, u)
            return carry

        lax.fori_loop(0, NCHL // 4, _outer, 0)

    @pl.when(cid == 0)
    def _():
        _main(slp, tlp, vlp, True)

    @pl.when(cid == 1)
    def _():
        _main(shp, thp, vhp, False)

    plsc.subcore_barrier()

    for c in range(ROWS_PT // C):
        r0 = start + c * C
        pltpu.sync_copy(z_sh.at[pl.ds(r0, C)], out_hbm.at[cid, pl.ds(r0, C)])


_sc_spmm = functools.partial(
    pl.kernel,
    out_type=jax.ShapeDtypeStruct((NC, NP, D), jnp.float32),
    mesh=plsc.VectorSubcoreMesh(core_axis_name="c", subcore_axis_name="s",
                                num_cores=NC, num_subcores=NS),
    compiler_params=pltpu.CompilerParams(needs_layout_passes=False),
    scratch_types=[
        pltpu.VMEM((C, D), jnp.float32),
        pltpu.VMEM((C, D), jnp.float32),
        pltpu.VMEM((C, D), jnp.float32),
        pltpu.VMEM((C, D), jnp.float32),
        pltpu.VMEM((4, C), jnp.int32),
        pltpu.VMEM((4, C), jnp.int32),
        pltpu.VMEM((4, C), jnp.float32),
        pltpu.VMEM((4, C), jnp.float32),
        pltpu.VMEM((4, C), jnp.int32),
        pltpu.VMEM((C,), jnp.float32),
        pltpu.SemaphoreType.DMA,
        pltpu.SemaphoreType.DMA,
        pltpu.SemaphoreType.DMA,
        pltpu.SemaphoreType.DMA,
        pltpu.VMEM_SHARED((NP, D), jnp.float32),
        pltpu.VMEM_SHARED((N,), jnp.float32),
    ],
)(_sc_spmm_body)


def _alpha_body(x_ref, tw_ref, tb_ref, o_ref):
    t = jnp.sum(x_ref[...] * tw_ref[...], axis=1, keepdims=True) + tb_ref[0, 0]
    o_ref[...] = 1.0 / (1.0 + jnp.exp(-t))


def _alpha_tc(x, theta_w, theta_b):
    blk = 400
    return pl.pallas_call(
        _alpha_body,
        grid=(N // blk,),
        in_specs=[
            pl.BlockSpec((blk, D), lambda i: (i, 0)),
            pl.BlockSpec((1, D), lambda i: (0, 0)),
            pl.BlockSpec((1, 1), lambda i: (0, 0)),
        ],
        out_specs=pl.BlockSpec((blk, 1), lambda i: (i, 0)),
        out_shape=jax.ShapeDtypeStruct((N, 1), jnp.float32),
    )(x, theta_w.reshape(1, D), theta_b.reshape(1, 1))


def _out_body(p_ref, w_ref, b_ref, o_ref):
    z = p_ref[0] + p_ref[1]
    o_ref[...] = jnp.maximum(
        jnp.dot(z, w_ref[...], preferred_element_type=jnp.float32) + b_ref[...],
        0.0)


def _out_tc(parts, W, b):
    blk = 2000
    return pl.pallas_call(
        _out_body,
        grid=(N // blk,),
        in_specs=[
            pl.BlockSpec((NC, blk, D), lambda i: (0, i, 0)),
            pl.BlockSpec((D, D), lambda i: (0, 0)),
            pl.BlockSpec((1, D), lambda i: (0, 0)),
        ],
        out_specs=pl.BlockSpec((blk, D), lambda i: (i, 0)),
        out_shape=jax.ShapeDtypeStruct((N, D), jnp.float32),
    )(parts, W, b.reshape(1, D))


def kernel(x, theta_w, theta_b, W, b, vals_lp, src_lp, dst_lp,
           vals_hp, src_hp, dst_hp):
    alpha = _alpha_tc(x, theta_w, theta_b)
    shp3 = (NS, NCHR, C)
    parts = _sc_spmm(x, alpha.reshape(N),
                     src_lp.reshape(shp3), dst_lp.reshape(shp3),
                     vals_lp.reshape(shp3),
                     src_hp.reshape(shp3), dst_hp.reshape(shp3),
                     vals_hp.reshape(shp3))
    out = _out_tc(parts, W, b)
    return out, alpha
